# baseline (speedup 1.0000x reference)
import jax
import jax.numpy as jnp
from jax import lax
from jax.experimental import pallas as pl
from jax.experimental.pallas import tpu as pltpu, tpu_sc as plsc

B = 16384
D = 64
LANES = 16
NUM_CORES = 2
NUM_SUBCORES = 16
NW = NUM_CORES * NUM_SUBCORES
BW = B // NW
GROUPS = BW // LANES
SEGS = D // LANES
PHASE = 256
PHASES = BW // PHASE

_MESH = dict(core_axis_name="c", subcore_axis_name="s",
             num_cores=NUM_CORES, num_subcores=NUM_SUBCORES)


def _bias_body(user_idx, item_idx, ubias8, ibias8, out,
               idx_u, idx_i, idx_u8, idx_i8, ub8, ib8, out_v,
               sem_ub, sem_ib):
    wid = lax.axis_index("s") * NUM_CORES + lax.axis_index("c")
    base = wid * BW

    pltpu.sync_copy(user_idx.at[pl.ds(base, BW)], idx_u)
    pltpu.sync_copy(item_idx.at[pl.ds(base, BW)], idx_i)

    def shift_chunk(c, carry):
        sl = pl.ds(c * LANES, LANES)
        idx_u8[sl] = jax.lax.shift_right_logical(idx_u[sl], 3)
        idx_i8[sl] = jax.lax.shift_right_logical(idx_i[sl], 3)
        return carry

    lax.fori_loop(0, BW // LANES, shift_chunk, 0)

    cub = pltpu.async_copy(ubias8.at[idx_u8], ub8, sem_ub)
    cib = pltpu.async_copy(ibias8.at[idx_i8], ib8, sem_ib)
    cub.wait()
    cib.wait()

    lanes = lax.iota(jnp.int32, LANES)

    def group(g, carry):
        r0 = g * LANES
        sl = pl.ds(r0, LANES)
        uv = idx_u[sl]
        iv = idx_i[sl]
        out_v[sl] = (plsc.load_gather(ub8, [r0 + lanes, uv & 7]) +
                     plsc.load_gather(ib8, [r0 + lanes, iv & 7]))
        return carry

    lax.fori_loop(0, GROUPS, group, 0)
    pltpu.sync_copy(out_v, out.at[pl.ds(base, BW)])


def _main_body(user_idx, item_idx, uemb2, iemb2, bias_sum, out,
               idx_u, idx_i, idx_u2, idx_i2, u_rows, i_rows, bias_v, out_v,
               sem_u, sem_i):
    wid = lax.axis_index("s") * NUM_CORES + lax.axis_index("c")
    base = wid * BW

    pltpu.sync_copy(user_idx.at[pl.ds(base, BW)], idx_u)
    pltpu.sync_copy(item_idx.at[pl.ds(base, BW)], idx_i)
    pltpu.sync_copy(bias_sum.at[pl.ds(base, BW)], bias_v)

    def shift_chunk(c, carry):
        sl = pl.ds(c * LANES, LANES)
        idx_u2[sl] = jax.lax.shift_right_logical(idx_u[sl], 1)
        idx_i2[sl] = jax.lax.shift_right_logical(idx_i[sl], 1)
        return carry

    lax.fori_loop(0, BW // LANES, shift_chunk, 0)

    lanes = lax.iota(jnp.int32, LANES)

    for ph in range(PHASES):
        p0 = ph * PHASE
        cu = pltpu.async_copy(
            uemb2.at[idx_u2.at[pl.ds(p0, PHASE)]], u_rows, sem_u)
        ci = pltpu.async_copy(
            iemb2.at[idx_i2.at[pl.ds(p0, PHASE)]], i_rows, sem_i)
        cu.wait()
        ci.wait()

        def group(g, carry):
            r0 = g * LANES
            sl = pl.ds(p0 + r0, LANES)
            uv = idx_u[sl]
            iv = idx_i[sl]
            acc = bias_v[sl]
            for r2 in range(LANES):
                r = r0 + r2
                ou = (uv[r2] & 1) * D
                oi = (iv[r2] & 1) * D
                p = (u_rows[r, pl.ds(ou, LANES)] *
                     i_rows[r, pl.ds(oi, LANES)])
                for j in range(1, SEGS):
                    p = p + (u_rows[r, pl.ds(ou + j * LANES, LANES)] *
                             i_rows[r, pl.ds(oi + j * LANES, LANES)])
                acc = acc + jnp.where(lanes == r2, jnp.sum(p), 0.0)
            out_v[sl] = acc
            return carry

        lax.fori_loop(0, PHASE // LANES, group, 0)

    pltpu.sync_copy(out_v, out.at[pl.ds(base, BW)])


def kernel(user, item, user_embeddings, item_embeddings, user_biases, item_biases):
    bias_call = pl.kernel(
        _bias_body,
        out_type=jax.ShapeDtypeStruct((B,), jnp.float32),
        compiler_params=pltpu.CompilerParams(needs_layout_passes=False,
                                             use_tc_tiling_on_sc=False),
        mesh=plsc.VectorSubcoreMesh(**_MESH),
        scratch_types=[
            pltpu.VMEM((BW,), jnp.int32),
            pltpu.VMEM((BW,), jnp.int32),
            pltpu.VMEM((BW,), jnp.int32),
            pltpu.VMEM((BW,), jnp.int32),
            pltpu.VMEM((BW, 8), jnp.float32),
            pltpu.VMEM((BW, 8), jnp.float32),
            pltpu.VMEM((BW,), jnp.float32),
            pltpu.SemaphoreType.DMA,
            pltpu.SemaphoreType.DMA,
        ],
    )
    bias_sum = bias_call(user, item,
                         user_biases.reshape(-1, 8), item_biases.reshape(-1, 8))

    main_call = pl.kernel(
        _main_body,
        out_type=jax.ShapeDtypeStruct((B,), jnp.float32),
        compiler_params=pltpu.CompilerParams(needs_layout_passes=False,
                                             use_tc_tiling_on_sc=True),
        mesh=plsc.VectorSubcoreMesh(**_MESH),
        scratch_types=[
            pltpu.VMEM((BW,), jnp.int32),
            pltpu.VMEM((BW,), jnp.int32),
            pltpu.VMEM((BW,), jnp.int32),
            pltpu.VMEM((BW,), jnp.int32),
            pltpu.VMEM((PHASE, 2 * D), jnp.float32),
            pltpu.VMEM((PHASE, 2 * D), jnp.float32),
            pltpu.VMEM((BW,), jnp.float32),
            pltpu.VMEM((BW,), jnp.float32),
            pltpu.SemaphoreType.DMA,
            pltpu.SemaphoreType.DMA,
        ],
    )
    return main_call(user, item,
                     user_embeddings.reshape(-1, 2 * D),
                     item_embeddings.reshape(-1, 2 * D),
                     bias_sum)


# speedup vs baseline: 1.3588x; 1.3588x over previous
import jax
import jax.numpy as jnp
from jax import lax
from jax.experimental import pallas as pl
from jax.experimental.pallas import tpu as pltpu, tpu_sc as plsc

B = 16384
D = 64
LANES = 16
NUM_CORES = 2
NUM_SUBCORES = 16
NW = NUM_CORES * NUM_SUBCORES
BW = B // NW
GROUPS = BW // LANES
SEGS = D // LANES
CHUNKS = BW // LANES

_MESH = dict(core_axis_name="c", subcore_axis_name="s",
             num_cores=NUM_CORES, num_subcores=NUM_SUBCORES)


def _bias_body(user_idx, item_idx, ubias8, ibias8, out,
               idx_u, idx_i, idx_u8, idx_i8, ub8, ib8, out_v,
               sem_ub, sem_ib):
    wid = lax.axis_index("s") * NUM_CORES + lax.axis_index("c")
    base = wid * BW

    pltpu.sync_copy(user_idx.at[pl.ds(base, BW)], idx_u)
    pltpu.sync_copy(item_idx.at[pl.ds(base, BW)], idx_i)

    def shift_chunk(c, carry):
        sl = pl.ds(c * LANES, LANES)
        idx_u8[sl] = jax.lax.shift_right_logical(idx_u[sl], 3)
        idx_i8[sl] = jax.lax.shift_right_logical(idx_i[sl], 3)
        return carry

    lax.fori_loop(0, BW // LANES, shift_chunk, 0)

    cub = pltpu.async_copy(ubias8.at[idx_u8], ub8, sem_ub)
    cib = pltpu.async_copy(ibias8.at[idx_i8], ib8, sem_ib)
    cub.wait()
    cib.wait()

    lanes = lax.iota(jnp.int32, LANES)

    def group(g, carry):
        r0 = g * LANES
        sl = pl.ds(r0, LANES)
        uv = idx_u[sl]
        iv = idx_i[sl]
        out_v[sl] = (plsc.load_gather(ub8, [r0 + lanes, uv & 7]) +
                     plsc.load_gather(ib8, [r0 + lanes, iv & 7]))
        return carry

    lax.fori_loop(0, GROUPS, group, 0)
    pltpu.sync_copy(out_v, out.at[pl.ds(base, BW)])


def _main_body(user_idx, item_idx, uemb, iemb, bias_sum, out,
               idx_u, idx_i, bias_v, out_v,
               ublk_a, iblk_a, ublk_b, iblk_b,
               sem_a, sem_b):
    wid = lax.axis_index("s") * NUM_CORES + lax.axis_index("c")
    base = wid * BW

    pltpu.sync_copy(user_idx.at[pl.ds(base, BW)], idx_u)
    pltpu.sync_copy(item_idx.at[pl.ds(base, BW)], idx_i)
    pltpu.sync_copy(bias_sum.at[pl.ds(base, BW)], bias_v)

    lanes = lax.iota(jnp.int32, LANES)

    def issue(c, ublk, iblk, sem):
        sl = pl.ds(c * LANES, LANES)
        uv = idx_u[sl]
        iv = idx_i[sl]
        for r2 in range(LANES):
            ua = pl.multiple_of((uv[r2] >> 3) * 8, 8)
            ia = pl.multiple_of((iv[r2] >> 3) * 8, 8)
            pltpu.async_copy(uemb.at[pl.ds(ua, 8), :], ublk.at[r2], sem)
            pltpu.async_copy(iemb.at[pl.ds(ia, 8), :], iblk.at[r2], sem)
        return uv, iv

    def drain(ublk, iblk, sem):
        for r2 in range(LANES):
            pltpu.make_async_copy(uemb.at[pl.ds(0, 8), :], ublk.at[r2],
                                  sem).wait()
            pltpu.make_async_copy(iemb.at[pl.ds(0, 8), :], iblk.at[r2],
                                  sem).wait()

    def compute(c, uv, iv, ublk, iblk):
        acc = bias_v[pl.ds(c * LANES, LANES)]
        for r2 in range(LANES):
            su = uv[r2] & 7
            si = iv[r2] & 7
            p = (ublk[r2, su, pl.ds(0, LANES)] *
                 iblk[r2, si, pl.ds(0, LANES)])
            for j in range(1, SEGS):
                p = p + (ublk[r2, su, pl.ds(j * LANES, LANES)] *
                         iblk[r2, si, pl.ds(j * LANES, LANES)])
            acc = acc + jnp.where(lanes == r2, jnp.sum(p), 0.0)
        out_v[pl.ds(c * LANES, LANES)] = acc

    issue(0, ublk_a, iblk_a, sem_a)

    def pipelined(d, carry):
        c_even = 2 * d
        uv_b, iv_b = issue(c_even + 1, ublk_b, iblk_b, sem_b)
        sl = pl.ds(c_even * LANES, LANES)
        uv_a = idx_u[sl]
        iv_a = idx_i[sl]
        drain(ublk_a, iblk_a, sem_a)
        compute(c_even, uv_a, iv_a, ublk_a, iblk_a)

        @pl.when(d < CHUNKS // 2 - 1)
        def _():
            issue(c_even + 2, ublk_a, iblk_a, sem_a)

        drain(ublk_b, iblk_b, sem_b)
        compute(c_even + 1, uv_b, iv_b, ublk_b, iblk_b)
        return carry

    lax.fori_loop(0, CHUNKS // 2, pipelined, 0)
    pltpu.sync_copy(out_v, out.at[pl.ds(base, BW)])


def kernel(user, item, user_embeddings, item_embeddings, user_biases, item_biases):
    bias_call = pl.kernel(
        _bias_body,
        out_type=jax.ShapeDtypeStruct((B,), jnp.float32),
        compiler_params=pltpu.CompilerParams(needs_layout_passes=False,
                                             use_tc_tiling_on_sc=False),
        mesh=plsc.VectorSubcoreMesh(**_MESH),
        scratch_types=[
            pltpu.VMEM((BW,), jnp.int32),
            pltpu.VMEM((BW,), jnp.int32),
            pltpu.VMEM((BW,), jnp.int32),
            pltpu.VMEM((BW,), jnp.int32),
            pltpu.VMEM((BW, 8), jnp.float32),
            pltpu.VMEM((BW, 8), jnp.float32),
            pltpu.VMEM((BW,), jnp.float32),
            pltpu.SemaphoreType.DMA,
            pltpu.SemaphoreType.DMA,
        ],
    )
    bias_sum = bias_call(user, item,
                         user_biases.reshape(-1, 8), item_biases.reshape(-1, 8))

    main_call = pl.kernel(
        _main_body,
        out_type=jax.ShapeDtypeStruct((B,), jnp.float32),
        compiler_params=pltpu.CompilerParams(needs_layout_passes=False,
                                             use_tc_tiling_on_sc=True),
        mesh=plsc.VectorSubcoreMesh(**_MESH),
        scratch_types=[
            pltpu.VMEM((BW,), jnp.int32),
            pltpu.VMEM((BW,), jnp.int32),
            pltpu.VMEM((BW,), jnp.float32),
            pltpu.VMEM((BW,), jnp.float32),
            pltpu.VMEM((LANES, 8, D), jnp.float32),
            pltpu.VMEM((LANES, 8, D), jnp.float32),
            pltpu.VMEM((LANES, 8, D), jnp.float32),
            pltpu.VMEM((LANES, 8, D), jnp.float32),
            pltpu.SemaphoreType.DMA,
            pltpu.SemaphoreType.DMA,
        ],
    )
    return main_call(user, item, user_embeddings, item_embeddings, bias_sum)


# speedup vs baseline: 1.9162x; 1.4103x over previous
import jax
import jax.numpy as jnp
from jax import lax
from jax.experimental import pallas as pl
from jax.experimental.pallas import tpu as pltpu, tpu_sc as plsc

B = 16384
D = 64
LANES = 16
NUM_CORES = 2
NUM_SUBCORES = 16
NW = NUM_CORES * NUM_SUBCORES
BW = B // NW
GROUPS = BW // LANES
SEGS = D // LANES
CQ = 4
CHUNKS = BW // CQ
FT = D // 8

_MESH = dict(core_axis_name="c", subcore_axis_name="s",
             num_cores=NUM_CORES, num_subcores=NUM_SUBCORES)


def _bias_body(user_idx, item_idx, ubias8, ibias8, out,
               idx_u, idx_i, idx_u8, idx_i8, ub8, ib8, out_v,
               sem_ub, sem_ib):
    wid = lax.axis_index("s") * NUM_CORES + lax.axis_index("c")
    base = wid * BW

    pltpu.sync_copy(user_idx.at[pl.ds(base, BW)], idx_u)
    pltpu.sync_copy(item_idx.at[pl.ds(base, BW)], idx_i)

    def shift_chunk(c, carry):
        sl = pl.ds(c * LANES, LANES)
        idx_u8[sl] = jax.lax.shift_right_logical(idx_u[sl], 3)
        idx_i8[sl] = jax.lax.shift_right_logical(idx_i[sl], 3)
        return carry

    lax.fori_loop(0, BW // LANES, shift_chunk, 0)

    cub = pltpu.async_copy(ubias8.at[idx_u8], ub8, sem_ub)
    cib = pltpu.async_copy(ibias8.at[idx_i8], ib8, sem_ib)
    cub.wait()
    cib.wait()

    lanes = lax.iota(jnp.int32, LANES)

    def group(g, carry):
        r0 = g * LANES
        sl = pl.ds(r0, LANES)
        uv = idx_u[sl]
        iv = idx_i[sl]
        out_v[sl] = (plsc.load_gather(ub8, [r0 + lanes, uv & 7]) +
                     plsc.load_gather(ib8, [r0 + lanes, iv & 7]))
        return carry

    lax.fori_loop(0, GROUPS, group, 0)
    pltpu.sync_copy(out_v, out.at[pl.ds(base, BW)])


def _main_body(user_idx, item_idx, uemb_t, iemb, bias_sum, out,
               idx_u, idx_i, out_v,
               ublk_a, iblk_a, ublk_b, iblk_b,
               sem_a, sem_b):
    wid = lax.axis_index("s") * NUM_CORES + lax.axis_index("c")
    base = wid * BW

    pltpu.sync_copy(user_idx.at[pl.ds(base, BW)], idx_u.at[pl.ds(0, BW)])
    pltpu.sync_copy(item_idx.at[pl.ds(base, BW)], idx_i.at[pl.ds(0, BW)])
    pltpu.sync_copy(bias_sum.at[pl.ds(base, BW)], out_v)

    lanes = lax.iota(jnp.int32, LANES)

    def issue(c, ublk, iblk, sem):
        uv = idx_u[pl.ds(c * CQ, LANES)]
        iv = idx_i[pl.ds(c * CQ, LANES)]
        for q in range(CQ):
            ut = pl.multiple_of((uv[q] >> 7) * 128, 128)
            ia = pl.multiple_of((iv[q] >> 3) * 8, 8)
            for ft in range(FT):
                pltpu.async_copy(
                    uemb_t.at[pl.ds(ft * 8, 8), pl.ds(ut, 128)],
                    ublk.at[q, ft], sem)
            pltpu.async_copy(iemb.at[pl.ds(ia, 8), :], iblk.at[q], sem)
        return uv, iv

    def drain(ublk, iblk, sem):
        for q in range(CQ):
            for ft in range(FT):
                pltpu.make_async_copy(
                    uemb_t.at[pl.ds(0, 8), pl.ds(0, 128)],
                    ublk.at[q, ft], sem).wait()
            pltpu.make_async_copy(iemb.at[pl.ds(0, 8), :],
                                  iblk.at[q], sem).wait()

    def compute(c, uv, iv, ublk, iblk):
        g16 = (c >> 2) * LANES
        ov = out_v[pl.ds(g16, LANES)]
        for q in range(CQ):
            ucol = lanes * 0 + (uv[q] & 127)
            si = iv[q] & 7
            acc16 = None
            for j in range(SEGS):
                fts = jax.lax.shift_right_logical(lanes + j * LANES, 3)
                svs = (lanes + j * LANES) & 7
                useg = plsc.load_gather(
                    ublk, [lanes * 0 + q, fts, svs, ucol])
                iseg = iblk[q, si, pl.ds(j * LANES, LANES)]
                prod = useg * iseg
                acc16 = prod if acc16 is None else acc16 + prod
            pos = (c & 3) * CQ + q
            ov = ov + jnp.where(lanes == pos, jnp.sum(acc16), 0.0)
        out_v[pl.ds(g16, LANES)] = ov

    issue(0, ublk_a, iblk_a, sem_a)

    def pipelined(d, carry):
        c_even = 2 * d
        uv_b, iv_b = issue(c_even + 1, ublk_b, iblk_b, sem_b)
        uv_a = idx_u[pl.ds(c_even * CQ, LANES)]
        iv_a = idx_i[pl.ds(c_even * CQ, LANES)]
        drain(ublk_a, iblk_a, sem_a)
        compute(c_even, uv_a, iv_a, ublk_a, iblk_a)

        @pl.when(d < CHUNKS // 2 - 1)
        def _():
            issue(c_even + 2, ublk_a, iblk_a, sem_a)

        drain(ublk_b, iblk_b, sem_b)
        compute(c_even + 1, uv_b, iv_b, ublk_b, iblk_b)
        return carry

    lax.fori_loop(0, CHUNKS // 2, pipelined, 0)
    pltpu.sync_copy(out_v, out.at[pl.ds(base, BW)])


def kernel(user, item, user_embeddings, item_embeddings, user_biases, item_biases):
    bias_call = pl.kernel(
        _bias_body,
        out_type=jax.ShapeDtypeStruct((B,), jnp.float32),
        compiler_params=pltpu.CompilerParams(needs_layout_passes=False,
                                             use_tc_tiling_on_sc=False),
        mesh=plsc.VectorSubcoreMesh(**_MESH),
        scratch_types=[
            pltpu.VMEM((BW,), jnp.int32),
            pltpu.VMEM((BW,), jnp.int32),
            pltpu.VMEM((BW,), jnp.int32),
            pltpu.VMEM((BW,), jnp.int32),
            pltpu.VMEM((BW, 8), jnp.float32),
            pltpu.VMEM((BW, 8), jnp.float32),
            pltpu.VMEM((BW,), jnp.float32),
            pltpu.SemaphoreType.DMA,
            pltpu.SemaphoreType.DMA,
        ],
    )
    bias_sum = bias_call(user, item,
                         user_biases.reshape(-1, 8), item_biases.reshape(-1, 8))

    main_call = pl.kernel(
        _main_body,
        out_type=jax.ShapeDtypeStruct((B,), jnp.float32),
        compiler_params=pltpu.CompilerParams(needs_layout_passes=False,
                                             use_tc_tiling_on_sc=True),
        mesh=plsc.VectorSubcoreMesh(**_MESH),
        scratch_types=[
            pltpu.VMEM((BW + LANES,), jnp.int32),
            pltpu.VMEM((BW + LANES,), jnp.int32),
            pltpu.VMEM((BW,), jnp.float32),
            pltpu.VMEM((CQ, FT, 8, 128), jnp.float32),
            pltpu.VMEM((CQ, 8, D), jnp.float32),
            pltpu.VMEM((CQ, FT, 8, 128), jnp.float32),
            pltpu.VMEM((CQ, 8, D), jnp.float32),
            pltpu.SemaphoreType.DMA,
            pltpu.SemaphoreType.DMA,
        ],
    )
    return main_call(user, item, user_embeddings.T, item_embeddings, bias_sum)


# speedup vs baseline: 2.1492x; 1.1216x over previous
import jax
import jax.numpy as jnp
from jax import lax
from jax.experimental import pallas as pl
from jax.experimental.pallas import tpu as pltpu, tpu_sc as plsc

B = 16384
D = 64
LANES = 16
NUM_CORES = 2
NUM_SUBCORES = 16
NW = NUM_CORES * NUM_SUBCORES
BW = B // NW
GROUPS = BW // LANES
SEGS = D // LANES
CQ = 4
CHUNKS = BW // CQ
FT = D // 8

_MESH = dict(core_axis_name="c", subcore_axis_name="s",
             num_cores=NUM_CORES, num_subcores=NUM_SUBCORES)


def _bias_body(user_idx, item_idx, ubias8, ibias8, dot, out,
               idx_u, idx_i, idx_u8, idx_i8, ub8, ib8, dot_v, out_v,
               sem_ub, sem_ib):
    wid = lax.axis_index("s") * NUM_CORES + lax.axis_index("c")
    base = wid * BW

    pltpu.sync_copy(user_idx.at[pl.ds(base, BW)], idx_u)
    pltpu.sync_copy(item_idx.at[pl.ds(base, BW)], idx_i)
    pltpu.sync_copy(dot.at[pl.ds(base, BW)], dot_v)

    def shift_chunk(c, carry):
        sl = pl.ds(c * LANES, LANES)
        idx_u8[sl] = jax.lax.shift_right_logical(idx_u[sl], 3)
        idx_i8[sl] = jax.lax.shift_right_logical(idx_i[sl], 3)
        return carry

    lax.fori_loop(0, BW // LANES, shift_chunk, 0)

    cub = pltpu.async_copy(ubias8.at[idx_u8], ub8, sem_ub)
    cib = pltpu.async_copy(ibias8.at[idx_i8], ib8, sem_ib)
    cub.wait()
    cib.wait()

    lanes = lax.iota(jnp.int32, LANES)

    def group(g, carry):
        r0 = g * LANES
        sl = pl.ds(r0, LANES)
        uv = idx_u[sl]
        iv = idx_i[sl]
        out_v[sl] = (dot_v[sl] +
                     plsc.load_gather(ub8, [r0 + lanes, uv & 7]) +
                     plsc.load_gather(ib8, [r0 + lanes, iv & 7]))
        return carry

    lax.fori_loop(0, GROUPS, group, 0)
    pltpu.sync_copy(out_v, out.at[pl.ds(base, BW)])


def _main_body(user_idx, item_idx, uemb_t, iemb, out,
               idx_u, idx_i, out_v,
               ublk_a, iblk_a, ublk_b, iblk_b,
               sem_a, sem_b):
    wid = lax.axis_index("s") * NUM_CORES + lax.axis_index("c")
    base = wid * BW

    pltpu.sync_copy(user_idx.at[pl.ds(base, BW)], idx_u.at[pl.ds(0, BW)])
    pltpu.sync_copy(item_idx.at[pl.ds(base, BW)], idx_i.at[pl.ds(0, BW)])

    lanes = lax.iota(jnp.int32, LANES)

    def zero_group(g, carry):
        out_v[pl.ds(g * LANES, LANES)] = lanes * 0.0
        return carry

    lax.fori_loop(0, GROUPS, zero_group, 0)

    def issue(c, ublk, iblk, sem):
        uv = idx_u[pl.ds(c * CQ, LANES)]
        iv = idx_i[pl.ds(c * CQ, LANES)]
        for q in range(CQ):
            ut = pl.multiple_of((uv[q] >> 7) * 128, 128)
            ia = pl.multiple_of((iv[q] >> 3) * 8, 8)
            for ft in range(FT):
                pltpu.async_copy(
                    uemb_t.at[pl.ds(ft * 8, 8), pl.ds(ut, 128)],
                    ublk.at[q, ft], sem)
            pltpu.async_copy(iemb.at[pl.ds(ia, 8), :], iblk.at[q], sem)
        return uv, iv

    def drain(ublk, iblk, sem):
        for q in range(CQ):
            for ft in range(FT):
                pltpu.make_async_copy(
                    uemb_t.at[pl.ds(0, 8), pl.ds(0, 128)],
                    ublk.at[q, ft], sem).wait()
            pltpu.make_async_copy(iemb.at[pl.ds(0, 8), :],
                                  iblk.at[q], sem).wait()

    def compute(c, uv, iv, ublk, iblk):
        g16 = (c >> 2) * LANES
        ov = out_v[pl.ds(g16, LANES)]
        for q in range(CQ):
            ucol = lanes * 0 + (uv[q] & 127)
            si = iv[q] & 7
            acc16 = None
            for j in range(SEGS):
                fts = jax.lax.shift_right_logical(lanes + j * LANES, 3)
                svs = (lanes + j * LANES) & 7
                useg = plsc.load_gather(
                    ublk, [lanes * 0 + q, fts, svs, ucol])
                iseg = iblk[q, si, pl.ds(j * LANES, LANES)]
                prod = useg * iseg
                acc16 = prod if acc16 is None else acc16 + prod
            pos = (c & 3) * CQ + q
            ov = ov + jnp.where(lanes == pos, jnp.sum(acc16), 0.0)
        out_v[pl.ds(g16, LANES)] = ov

    issue(0, ublk_a, iblk_a, sem_a)

    def pipelined(d, carry):
        c_even = 2 * d
        uv_b, iv_b = issue(c_even + 1, ublk_b, iblk_b, sem_b)
        uv_a = idx_u[pl.ds(c_even * CQ, LANES)]
        iv_a = idx_i[pl.ds(c_even * CQ, LANES)]
        drain(ublk_a, iblk_a, sem_a)
        compute(c_even, uv_a, iv_a, ublk_a, iblk_a)

        @pl.when(d < CHUNKS // 2 - 1)
        def _():
            issue(c_even + 2, ublk_a, iblk_a, sem_a)

        drain(ublk_b, iblk_b, sem_b)
        compute(c_even + 1, uv_b, iv_b, ublk_b, iblk_b)
        return carry

    lax.fori_loop(0, CHUNKS // 2, pipelined, 0)
    pltpu.sync_copy(out_v, out.at[pl.ds(base, BW)])


def kernel(user, item, user_embeddings, item_embeddings, user_biases, item_biases):
    bias_call = pl.kernel(
        _bias_body,
        out_type=jax.ShapeDtypeStruct((B,), jnp.float32),
        compiler_params=pltpu.CompilerParams(needs_layout_passes=False,
                                             use_tc_tiling_on_sc=False),
        mesh=plsc.VectorSubcoreMesh(**_MESH),
        scratch_types=[
            pltpu.VMEM((BW,), jnp.int32),
            pltpu.VMEM((BW,), jnp.int32),
            pltpu.VMEM((BW,), jnp.int32),
            pltpu.VMEM((BW,), jnp.int32),
            pltpu.VMEM((BW, 8), jnp.float32),
            pltpu.VMEM((BW, 8), jnp.float32),
            pltpu.VMEM((BW,), jnp.float32),
            pltpu.VMEM((BW,), jnp.float32),
            pltpu.SemaphoreType.DMA,
            pltpu.SemaphoreType.DMA,
        ],
    )

    main_call = pl.kernel(
        _main_body,
        out_type=jax.ShapeDtypeStruct((B,), jnp.float32),
        compiler_params=pltpu.CompilerParams(needs_layout_passes=False,
                                             use_tc_tiling_on_sc=True),
        mesh=plsc.VectorSubcoreMesh(**_MESH),
        scratch_types=[
            pltpu.VMEM((BW + LANES,), jnp.int32),
            pltpu.VMEM((BW + LANES,), jnp.int32),
            pltpu.VMEM((BW,), jnp.float32),
            pltpu.VMEM((CQ, FT, 8, 128), jnp.float32),
            pltpu.VMEM((CQ, 8, D), jnp.float32),
            pltpu.VMEM((CQ, FT, 8, 128), jnp.float32),
            pltpu.VMEM((CQ, 8, D), jnp.float32),
            pltpu.SemaphoreType.DMA,
            pltpu.SemaphoreType.DMA,
        ],
    )
    dot = main_call(user, item, user_embeddings.T, item_embeddings)
    return bias_call(user, item,
                     user_biases.reshape(-1, 8), item_biases.reshape(-1, 8),
                     dot)


# speedup vs baseline: 2.1579x; 1.0041x over previous
import jax
import jax.numpy as jnp
from jax import lax
from jax.experimental import pallas as pl
from jax.experimental.pallas import tpu as pltpu, tpu_sc as plsc

B = 16384
D = 64
LANES = 16
NUM_CORES = 2
NUM_SUBCORES = 16
NW = NUM_CORES * NUM_SUBCORES
BW = B // NW
GROUPS = BW // LANES
SEGS = D // LANES
CQ = 4
CHUNKS = BW // CQ
FT = D // 8

_MESH = dict(core_axis_name="c", subcore_axis_name="s",
             num_cores=NUM_CORES, num_subcores=NUM_SUBCORES)


def _bias_body(user_idx, item_idx, ubias8, ibias8, dot, out,
               idx_u, idx_i, idx_u8, idx_i8, ub8, ib8, dot_v, out_v,
               sem_ub, sem_ib):
    wid = lax.axis_index("s") * NUM_CORES + lax.axis_index("c")
    base = wid * BW

    pltpu.sync_copy(user_idx.at[pl.ds(base, BW)], idx_u)
    pltpu.sync_copy(item_idx.at[pl.ds(base, BW)], idx_i)
    pltpu.sync_copy(dot.at[pl.ds(base, BW)], dot_v)

    def shift_chunk(c, carry):
        sl = pl.ds(c * LANES, LANES)
        idx_u8[sl] = jax.lax.shift_right_logical(idx_u[sl], 3)
        idx_i8[sl] = jax.lax.shift_right_logical(idx_i[sl], 3)
        return carry

    lax.fori_loop(0, BW // LANES, shift_chunk, 0)

    cub = pltpu.async_copy(ubias8.at[idx_u8], ub8, sem_ub)
    cib = pltpu.async_copy(ibias8.at[idx_i8], ib8, sem_ib)
    cub.wait()
    cib.wait()

    lanes = lax.iota(jnp.int32, LANES)

    def group(g, carry):
        r0 = g * LANES
        sl = pl.ds(r0, LANES)
        uv = idx_u[sl]
        iv = idx_i[sl]
        out_v[sl] = (dot_v[sl] +
                     plsc.load_gather(ub8, [r0 + lanes, uv & 7]) +
                     plsc.load_gather(ib8, [r0 + lanes, iv & 7]))
        return carry

    lax.fori_loop(0, GROUPS, group, 0)
    pltpu.sync_copy(out_v, out.at[pl.ds(base, BW)])


def _main_body(user_idx, item_idx, uemb_t, iemb, out,
               idx_u, idx_i, out_v,
               ublk_a, iblk_a, ublk_b, iblk_b,
               sem_a, sem_b):
    wid = lax.axis_index("s") * NUM_CORES + lax.axis_index("c")
    base = wid * BW

    pltpu.sync_copy(user_idx.at[pl.ds(base, BW)], idx_u.at[pl.ds(0, BW)])
    pltpu.sync_copy(item_idx.at[pl.ds(base, BW)], idx_i.at[pl.ds(0, BW)])

    lanes = lax.iota(jnp.int32, LANES)

    def zero_group(g, carry):
        out_v[pl.ds(g * LANES, LANES)] = lanes * 0.0
        return carry

    lax.fori_loop(0, GROUPS, zero_group, 0)

    def issue(c, ublk, iblk, sem):
        uv = idx_u[pl.ds(c * CQ, LANES)]
        iv = idx_i[pl.ds(c * CQ, LANES)]
        for q in range(CQ):
            ut = pl.multiple_of((uv[q] >> 7) * 128, 128)
            ia = pl.multiple_of((iv[q] >> 3) * 8, 8)
            pltpu.async_copy(uemb_t.at[:, pl.ds(ut, 128)], ublk.at[q], sem)
            pltpu.async_copy(iemb.at[pl.ds(ia, 8), :], iblk.at[q], sem)
        return uv, iv

    def drain(ublk, iblk, sem):
        for q in range(CQ):
            pltpu.make_async_copy(uemb_t.at[:, pl.ds(0, 128)],
                                  ublk.at[q], sem).wait()
            pltpu.make_async_copy(iemb.at[pl.ds(0, 8), :],
                                  iblk.at[q], sem).wait()

    def compute(c, uv, iv, ublk, iblk):
        g16 = (c >> 2) * LANES
        ov = out_v[pl.ds(g16, LANES)]
        for q in range(CQ):
            ucol = lanes * 0 + (uv[q] & 127)
            si = iv[q] & 7
            acc16 = None
            for j in range(SEGS):
                useg = plsc.load_gather(
                    ublk, [lanes * 0 + q, lanes + j * LANES, ucol])
                iseg = iblk[q, si, pl.ds(j * LANES, LANES)]
                prod = useg * iseg
                acc16 = prod if acc16 is None else acc16 + prod
            pos = (c & 3) * CQ + q
            ov = ov + jnp.where(lanes == pos, jnp.sum(acc16), 0.0)
        out_v[pl.ds(g16, LANES)] = ov

    issue(0, ublk_a, iblk_a, sem_a)

    def pipelined(d, carry):
        c_even = 2 * d
        uv_b, iv_b = issue(c_even + 1, ublk_b, iblk_b, sem_b)
        uv_a = idx_u[pl.ds(c_even * CQ, LANES)]
        iv_a = idx_i[pl.ds(c_even * CQ, LANES)]
        drain(ublk_a, iblk_a, sem_a)
        compute(c_even, uv_a, iv_a, ublk_a, iblk_a)

        @pl.when(d < CHUNKS // 2 - 1)
        def _():
            issue(c_even + 2, ublk_a, iblk_a, sem_a)

        drain(ublk_b, iblk_b, sem_b)
        compute(c_even + 1, uv_b, iv_b, ublk_b, iblk_b)
        return carry

    lax.fori_loop(0, CHUNKS // 2, pipelined, 0)
    pltpu.sync_copy(out_v, out.at[pl.ds(base, BW)])


def kernel(user, item, user_embeddings, item_embeddings, user_biases, item_biases):
    bias_call = pl.kernel(
        _bias_body,
        out_type=jax.ShapeDtypeStruct((B,), jnp.float32),
        compiler_params=pltpu.CompilerParams(needs_layout_passes=False,
                                             use_tc_tiling_on_sc=False),
        mesh=plsc.VectorSubcoreMesh(**_MESH),
        scratch_types=[
            pltpu.VMEM((BW,), jnp.int32),
            pltpu.VMEM((BW,), jnp.int32),
            pltpu.VMEM((BW,), jnp.int32),
            pltpu.VMEM((BW,), jnp.int32),
            pltpu.VMEM((BW, 8), jnp.float32),
            pltpu.VMEM((BW, 8), jnp.float32),
            pltpu.VMEM((BW,), jnp.float32),
            pltpu.VMEM((BW,), jnp.float32),
            pltpu.SemaphoreType.DMA,
            pltpu.SemaphoreType.DMA,
        ],
    )

    main_call = pl.kernel(
        _main_body,
        out_type=jax.ShapeDtypeStruct((B,), jnp.float32),
        compiler_params=pltpu.CompilerParams(needs_layout_passes=False,
                                             use_tc_tiling_on_sc=True),
        mesh=plsc.VectorSubcoreMesh(**_MESH),
        scratch_types=[
            pltpu.VMEM((BW + LANES,), jnp.int32),
            pltpu.VMEM((BW + LANES,), jnp.int32),
            pltpu.VMEM((BW,), jnp.float32),
            pltpu.VMEM((CQ, D, 128), jnp.float32),
            pltpu.VMEM((CQ, 8, D), jnp.float32),
            pltpu.VMEM((CQ, D, 128), jnp.float32),
            pltpu.VMEM((CQ, 8, D), jnp.float32),
            pltpu.SemaphoreType.DMA,
            pltpu.SemaphoreType.DMA,
        ],
    )
    dot = main_call(user, item, user_embeddings.T, item_embeddings)
    return bias_call(user, item,
                     user_biases.reshape(-1, 8), item_biases.reshape(-1, 8),
                     dot)
